# baseline (speedup 1.0000x reference)
import functools

import jax
import jax.numpy as jnp
from jax import lax
from jax.experimental import pallas as pl
from jax.experimental.pallas import tpu as pltpu
from jax.experimental.pallas import tpu_sc as plsc

N = 10000
E = 320000
D = 128
NC = 2
NS = 16
NW = NC * NS
CHUNK = 128
CHP = 80
CN = NW * CHP
E2 = CN * CHUNK
TRASH = 512
PAD = E2 - E
RPS = 624
TAIL = N - NS * RPS
TAIL_BASE = NS * RPS

_EPS_BN = 1e-5
_EPS_NORM = 1e-12

_mesh = plsc.VectorSubcoreMesh(core_axis_name="c", subcore_axis_name="s")


def _zero_stripe(zfeat, accum, s):
    base0 = s * RPS
    pltpu.sync_copy(zfeat.at[pl.ds(base0, RPS)], accum.at[pl.ds(base0, RPS)])

    @pl.when(s == NS - 1)
    def _tail():
        pltpu.sync_copy(zfeat.at[pl.ds(TAIL_BASE, TAIL)],
                        accum.at[pl.ds(TAIL_BASE, TAIL)])


def _write_stripe(accum, out, c, s):
    base0 = s * RPS
    pltpu.sync_copy(accum.at[pl.ds(base0, RPS)],
                    out.at[c, pl.ds(base0, RPS)])

    @pl.when(s == NS - 1)
    def _tail():
        pltpu.sync_copy(accum.at[pl.ds(TAIL_BASE, TAIL)],
                        out.at[c, pl.ds(TAIL_BASE, TAIL)])


CHPH = CHP // 2


def _edge_loop(table, srcp, dstp, w, sidx_big, didx_big, rows0, rows1,
               accum, sem0, sem1):
    for half in range(2):
        base = w * CHP + half * CHPH
        pltpu.sync_copy(srcp.at[pl.ds(base, CHPH)], sidx_big)
        pltpu.sync_copy(dstp.at[pl.ds(base, CHPH)], didx_big)

        def step(k, carry):
            pltpu.async_copy(table.at[sidx_big.at[k]], rows0, sem0).wait()
            pltpu.sync_copy(rows0, accum.at[didx_big.at[k]], add=True)
            return carry

        lax.fori_loop(0, CHPH, step, 0)


@functools.partial(
    pl.kernel,
    out_type=[
        jax.ShapeDtypeStruct((NC, N, D), jnp.float32),
        jax.ShapeDtypeStruct((NC, N, D), jnp.float32),
    ],
    mesh=_mesh,
    scratch_types=[
        pltpu.VMEM((CHPH, CHUNK), jnp.int32),
        pltpu.VMEM((CHPH, CHUNK), jnp.int32),
        pltpu.VMEM((CHUNK, D), jnp.float32),
        pltpu.VMEM((CHUNK, D), jnp.float32),
        pltpu.VMEM_SHARED((N + TRASH, D), jnp.float32),
        pltpu.SemaphoreType.DMA,
        pltpu.SemaphoreType.DMA,
    ],
)
def _agg_counts(table, srcp, dstp, zfeat, ones, sums_out, cnt_out,
                sidx_big, didx_big, rows0, rows1, accum, sem0, sem1):
    c = lax.axis_index("c")
    s = lax.axis_index("s")
    w = s * NC + c

    _zero_stripe(zfeat, accum, s)
    pltpu.sync_copy(ones, rows0)
    plsc.subcore_barrier()

    for half in range(2):
        pltpu.sync_copy(dstp.at[pl.ds(w * CHP + half * CHPH, CHPH)],
                        didx_big)

        def cgroup(g, carry):
            pltpu.sync_copy(rows0, accum.at[didx_big.at[g]], add=True)
            return carry

        lax.fori_loop(0, CHPH, cgroup, 0)

    plsc.subcore_barrier()
    _write_stripe(accum, cnt_out, c, s)

    _zero_stripe(zfeat, accum, s)
    plsc.subcore_barrier()
    _edge_loop(table, srcp, dstp, w, sidx_big, didx_big, rows0, rows1,
               accum, sem0, sem1)
    plsc.subcore_barrier()
    _write_stripe(accum, sums_out, c, s)


@functools.partial(
    pl.kernel,
    out_type=jax.ShapeDtypeStruct((NC, N, D), jnp.float32),
    mesh=_mesh,
    scratch_types=[
        pltpu.VMEM((CHPH, CHUNK), jnp.int32),
        pltpu.VMEM((CHPH, CHUNK), jnp.int32),
        pltpu.VMEM((CHUNK, D), jnp.float32),
        pltpu.VMEM((CHUNK, D), jnp.float32),
        pltpu.VMEM_SHARED((N + TRASH, D), jnp.float32),
        pltpu.SemaphoreType.DMA,
        pltpu.SemaphoreType.DMA,
    ],
)
def _agg(table, srcp, dstp, zfeat, sums_out,
         sidx_big, didx_big, rows0, rows1, accum, sem0, sem1):
    c = lax.axis_index("c")
    s = lax.axis_index("s")
    w = s * NC + c
    _zero_stripe(zfeat, accum, s)
    plsc.subcore_barrier()
    _edge_loop(table, srcp, dstp, w, sidx_big, didx_big, rows0, rows1,
               accum, sem0, sem1)
    plsc.subcore_barrier()
    _write_stripe(accum, sums_out, c, s)


_R = 1000


def _dense1_body(sp_ref, cp_ref, x_ref, wl_ref, bl_ref, wr_ref, g_ref, b_ref,
                 o_ref):
    ssum = sp_ref[0] + sp_ref[1]
    cnt = cp_ref[0][:, 0:1] + cp_ref[1][:, 0:1]
    mean = ssum / jnp.maximum(cnt, 1.0)
    out = (jnp.dot(mean, wl_ref[...], preferred_element_type=jnp.float32)
           + jnp.dot(x_ref[...], wr_ref[...], preferred_element_type=jnp.float32)
           + bl_ref[...])
    nrm = jnp.sqrt(jnp.sum(out * out, axis=1, keepdims=True))
    out = out / jnp.maximum(nrm, _EPS_NORM)
    out = jnp.maximum(out, 0.0)
    o_ref[...] = g_ref[...] * out * (1.0 / jnp.sqrt(1.0 + _EPS_BN)) + b_ref[...]


def _dense2_body(sp_ref, cp_ref, h_ref, wl_ref, bl_ref, wr_ref, wfc_ref,
                 bfc_ref, o_ref):
    ssum = sp_ref[0] + sp_ref[1]
    cnt = cp_ref[0][:, 0:1] + cp_ref[1][:, 0:1]
    mean = ssum / jnp.maximum(cnt, 1.0)
    out = (jnp.dot(mean, wl_ref[...], preferred_element_type=jnp.float32)
           + jnp.dot(h_ref[...], wr_ref[...], preferred_element_type=jnp.float32)
           + bl_ref[...])
    nrm = jnp.sqrt(jnp.sum(out * out, axis=1, keepdims=True))
    out = out / jnp.maximum(nrm, _EPS_NORM)
    o_ref[...] = (jnp.sum(out * wfc_ref[...], axis=1, keepdims=True)
                  + bfc_ref[...])


def _row_specs():
    return [
        pl.BlockSpec((NC, _R, D), lambda i: (0, i, 0)),
        pl.BlockSpec((NC, _R, D), lambda i: (0, i, 0)),
        pl.BlockSpec((_R, D), lambda i: (i, 0)),
    ]


def _full2d(shape):
    return pl.BlockSpec(shape, lambda i: (0, 0))


def _dense1(sp, cp, x, wl, bl, wr, g, b):
    return pl.pallas_call(
        _dense1_body,
        grid=(N // _R,),
        in_specs=_row_specs() + [
            _full2d((D, D)), _full2d((1, D)), _full2d((D, D)),
            _full2d((1, D)), _full2d((1, D)),
        ],
        out_specs=pl.BlockSpec((_R, D), lambda i: (i, 0)),
        out_shape=jax.ShapeDtypeStruct((N, D), jnp.float32),
    )(sp, cp, x, wl, bl, wr, g, b)


def _dense2(sp, cp, h, wl, bl, wr, wfc, bfc):
    return pl.pallas_call(
        _dense2_body,
        grid=(N // _R,),
        in_specs=_row_specs() + [
            _full2d((D, D)), _full2d((1, D)), _full2d((D, D)),
            _full2d((1, D)), _full2d((1, 1)),
        ],
        out_specs=pl.BlockSpec((_R, 1), lambda i: (i, 0)),
        out_shape=jax.ShapeDtypeStruct((N, 1), jnp.float32),
    )(sp, cp, h, wl, bl, wr, wfc, bfc)


def kernel(x, edge_index, W1l, b1l, W1r, gamma, beta, W2l, b2l, W2r, Wfc, bfc):
    src = edge_index[0]
    dst = edge_index[1]
    srcp = jnp.concatenate(
        [src, jnp.zeros((PAD,), jnp.int32)]).reshape(CN, CHUNK)
    dstp = jnp.concatenate(
        [dst, N + (jnp.arange(PAD, dtype=jnp.int32) % TRASH)]
    ).reshape(CN, CHUNK)
    zfeat = jnp.zeros((N, D), jnp.float32)
    ones = jnp.ones((CHUNK, D), jnp.float32)

    sums1, cnts = _agg_counts(x, srcp, dstp, zfeat, ones)
    h = _dense1(sums1, cnts, x, W1l, b1l.reshape(1, D), W1r,
                gamma.reshape(1, D), beta.reshape(1, D))
    sums2 = _agg(h, srcp, dstp, zfeat)
    out = _dense2(sums2, cnts, h, W2l, b2l.reshape(1, D), W2r,
                  Wfc.reshape(1, D), bfc.reshape(1, 1))
    return out.reshape(N)


# speedup vs baseline: 2.1426x; 2.1426x over previous
import functools

import jax
import jax.numpy as jnp
from jax import lax
from jax.experimental import pallas as pl
from jax.experimental.pallas import tpu as pltpu
from jax.experimental.pallas import tpu_sc as plsc

N = 10000
E = 320000
D = 128
NC = 2
NS = 16
NW = NC * NS
EPW = E // NW
CHUNK = 80
ITERS = EPW // CHUNK
RPS = 624
TAIL = N - NS * RPS
TAIL_BASE = NS * RPS

_EPS_BN = 1e-5
_EPS_NORM = 1e-12

_mesh = plsc.VectorSubcoreMesh(core_axis_name="c", subcore_axis_name="s")


def _zero_stripe(zfeat, accum, s):
    base0 = s * RPS
    pltpu.sync_copy(zfeat.at[pl.ds(base0, RPS)], accum.at[pl.ds(base0, RPS)])

    @pl.when(s == NS - 1)
    def _tail():
        pltpu.sync_copy(zfeat.at[pl.ds(TAIL_BASE, TAIL)],
                        accum.at[pl.ds(TAIL_BASE, TAIL)])


def _write_stripe(accum, out, c, s):
    base0 = s * RPS
    pltpu.sync_copy(accum.at[pl.ds(base0, RPS)],
                    out.at[c, pl.ds(base0, RPS)])

    @pl.when(s == NS - 1)
    def _tail():
        pltpu.sync_copy(accum.at[pl.ds(TAIL_BASE, TAIL)],
                        out.at[c, pl.ds(TAIL_BASE, TAIL)])


def _edge_loop(table, src, dst, wbase,
               sidx0, sidx1, didx0, didx1, rows0, rows1, accum, sem0, sem1):
    def load_idx(k, sidx, didx):
        base = wbase + k * CHUNK
        pltpu.sync_copy(src.at[pl.ds(base, CHUNK)], sidx)
        pltpu.sync_copy(dst.at[pl.ds(base, CHUNK)], didx)

    def pair(j0, carry):
        k0 = 2 * j0
        load_idx(k0 + 1, sidx1, didx1)
        pltpu.async_copy(table.at[sidx1], rows1, sem1)
        pltpu.make_async_copy(table.at[sidx0], rows0, sem0).wait()
        pltpu.sync_copy(rows0, accum.at[didx0], add=True)
        load_idx(k0 + 2, sidx0, didx0)
        pltpu.async_copy(table.at[sidx0], rows0, sem0)
        pltpu.make_async_copy(table.at[sidx1], rows1, sem1).wait()
        pltpu.sync_copy(rows1, accum.at[didx1], add=True)
        return carry

    load_idx(0, sidx0, didx0)
    pltpu.async_copy(table.at[sidx0], rows0, sem0)
    lax.fori_loop(0, (ITERS - 1) // 2, pair, 0)
    pltpu.make_async_copy(table.at[sidx0], rows0, sem0).wait()
    pltpu.sync_copy(rows0, accum.at[didx0], add=True)


@functools.partial(
    pl.kernel,
    out_type=[
        jax.ShapeDtypeStruct((NC, N, D), jnp.float32),
        jax.ShapeDtypeStruct((NC, N, D), jnp.float32),
    ],
    mesh=_mesh,
    scratch_types=[
        pltpu.VMEM((CHUNK,), jnp.int32),
        pltpu.VMEM((CHUNK,), jnp.int32),
        pltpu.VMEM((CHUNK,), jnp.int32),
        pltpu.VMEM((CHUNK,), jnp.int32),
        pltpu.VMEM((CHUNK, D), jnp.float32),
        pltpu.VMEM((CHUNK, D), jnp.float32),
        pltpu.VMEM_SHARED((N, D), jnp.float32),
        pltpu.SemaphoreType.DMA,
        pltpu.SemaphoreType.DMA,
    ],
)
def _agg_counts(table, src, dst, zfeat, ones, sums_out, cnt_out,
                sidx0, sidx1, didx0, didx1, rows0, rows1, accum, sem0, sem1):
    c = lax.axis_index("c")
    s = lax.axis_index("s")
    w = s * NC + c
    wbase = w * EPW

    _zero_stripe(zfeat, accum, s)
    pltpu.sync_copy(ones, rows0)
    plsc.subcore_barrier()

    def cpair(j0, carry):
        k0 = 2 * j0
        pltpu.sync_copy(dst.at[pl.ds(wbase + (k0 + 1) * CHUNK, CHUNK)],
                        didx1)
        pltpu.sync_copy(rows0, accum.at[didx0], add=True)
        pltpu.sync_copy(dst.at[pl.ds(wbase + (k0 + 2) * CHUNK, CHUNK)],
                        didx0)
        pltpu.sync_copy(rows0, accum.at[didx1], add=True)
        return carry

    pltpu.sync_copy(dst.at[pl.ds(wbase, CHUNK)], didx0)
    lax.fori_loop(0, (ITERS - 1) // 2, cpair, 0)
    pltpu.sync_copy(rows0, accum.at[didx0], add=True)
    plsc.subcore_barrier()
    _write_stripe(accum, cnt_out, c, s)

    _zero_stripe(zfeat, accum, s)
    plsc.subcore_barrier()
    _edge_loop(table, src, dst, wbase, sidx0, sidx1, didx0, didx1,
               rows0, rows1, accum, sem0, sem1)
    plsc.subcore_barrier()
    _write_stripe(accum, sums_out, c, s)


@functools.partial(
    pl.kernel,
    out_type=jax.ShapeDtypeStruct((NC, N, D), jnp.float32),
    mesh=_mesh,
    scratch_types=[
        pltpu.VMEM((CHUNK,), jnp.int32),
        pltpu.VMEM((CHUNK,), jnp.int32),
        pltpu.VMEM((CHUNK,), jnp.int32),
        pltpu.VMEM((CHUNK,), jnp.int32),
        pltpu.VMEM((CHUNK, D), jnp.float32),
        pltpu.VMEM((CHUNK, D), jnp.float32),
        pltpu.VMEM_SHARED((N, D), jnp.float32),
        pltpu.SemaphoreType.DMA,
        pltpu.SemaphoreType.DMA,
    ],
)
def _agg(table, src, dst, zfeat, sums_out,
         sidx0, sidx1, didx0, didx1, rows0, rows1, accum, sem0, sem1):
    c = lax.axis_index("c")
    s = lax.axis_index("s")
    w = s * NC + c
    _zero_stripe(zfeat, accum, s)
    plsc.subcore_barrier()
    _edge_loop(table, src, dst, w * EPW, sidx0, sidx1, didx0, didx1,
               rows0, rows1, accum, sem0, sem1)
    plsc.subcore_barrier()
    _write_stripe(accum, sums_out, c, s)


_R = 1000


def _dense1_body(sp_ref, cp_ref, x_ref, wl_ref, bl_ref, wr_ref, g_ref, b_ref,
                 o_ref):
    ssum = sp_ref[0] + sp_ref[1]
    cnt = cp_ref[0][:, 0:1] + cp_ref[1][:, 0:1]
    mean = ssum / jnp.maximum(cnt, 1.0)
    out = (jnp.dot(mean, wl_ref[...], preferred_element_type=jnp.float32)
           + jnp.dot(x_ref[...], wr_ref[...], preferred_element_type=jnp.float32)
           + bl_ref[...])
    nrm = jnp.sqrt(jnp.sum(out * out, axis=1, keepdims=True))
    out = out / jnp.maximum(nrm, _EPS_NORM)
    out = jnp.maximum(out, 0.0)
    o_ref[...] = g_ref[...] * out * (1.0 / jnp.sqrt(1.0 + _EPS_BN)) + b_ref[...]


def _dense2_body(sp_ref, cp_ref, h_ref, wl_ref, bl_ref, wr_ref, wfc_ref,
                 bfc_ref, o_ref):
    ssum = sp_ref[0] + sp_ref[1]
    cnt = cp_ref[0][:, 0:1] + cp_ref[1][:, 0:1]
    mean = ssum / jnp.maximum(cnt, 1.0)
    out = (jnp.dot(mean, wl_ref[...], preferred_element_type=jnp.float32)
           + jnp.dot(h_ref[...], wr_ref[...], preferred_element_type=jnp.float32)
           + bl_ref[...])
    nrm = jnp.sqrt(jnp.sum(out * out, axis=1, keepdims=True))
    out = out / jnp.maximum(nrm, _EPS_NORM)
    o_ref[...] = (jnp.sum(out * wfc_ref[...], axis=1, keepdims=True)
                  + bfc_ref[...])


def _row_specs():
    return [
        pl.BlockSpec((NC, _R, D), lambda i: (0, i, 0)),
        pl.BlockSpec((NC, _R, D), lambda i: (0, i, 0)),
        pl.BlockSpec((_R, D), lambda i: (i, 0)),
    ]


def _full2d(shape):
    return pl.BlockSpec(shape, lambda i: (0, 0))


def _dense1(sp, cp, x, wl, bl, wr, g, b):
    return pl.pallas_call(
        _dense1_body,
        grid=(N // _R,),
        in_specs=_row_specs() + [
            _full2d((D, D)), _full2d((1, D)), _full2d((D, D)),
            _full2d((1, D)), _full2d((1, D)),
        ],
        out_specs=pl.BlockSpec((_R, D), lambda i: (i, 0)),
        out_shape=jax.ShapeDtypeStruct((N, D), jnp.float32),
    )(sp, cp, x, wl, bl, wr, g, b)


def _dense2(sp, cp, h, wl, bl, wr, wfc, bfc):
    return pl.pallas_call(
        _dense2_body,
        grid=(N // _R,),
        in_specs=_row_specs() + [
            _full2d((D, D)), _full2d((1, D)), _full2d((D, D)),
            _full2d((1, D)), _full2d((1, 1)),
        ],
        out_specs=pl.BlockSpec((_R, 1), lambda i: (i, 0)),
        out_shape=jax.ShapeDtypeStruct((N, 1), jnp.float32),
    )(sp, cp, h, wl, bl, wr, wfc, bfc)


def kernel(x, edge_index, W1l, b1l, W1r, gamma, beta, W2l, b2l, W2r, Wfc, bfc):
    src = edge_index[0]
    dst = edge_index[1]
    zfeat = jnp.zeros((N, D), jnp.float32)
    ones = jnp.ones((CHUNK, D), jnp.float32)

    sums1, cnts = _agg_counts(x, src, dst, zfeat, ones)
    h = _dense1(sums1, cnts, x, W1l, b1l.reshape(1, D), W1r,
                gamma.reshape(1, D), beta.reshape(1, D))
    sums2 = _agg(h, src, dst, zfeat)
    out = _dense2(sums2, cnts, h, W2l, b2l.reshape(1, D), W2r,
                  Wfc.reshape(1, D), bfc.reshape(1, 1))
    return out.reshape(N)


# speedup vs baseline: 2.3681x; 1.1053x over previous
import functools

import jax
import jax.numpy as jnp
from jax import lax
from jax.experimental import pallas as pl
from jax.experimental.pallas import tpu as pltpu
from jax.experimental.pallas import tpu_sc as plsc

N = 10000
E = 320000
D = 128
NC = 2
NS = 16
NW = NC * NS
EPW = E // NW
CHUNK = 80
ITERS = EPW // CHUNK
RPS = 624
TAIL = N - NS * RPS
TAIL_BASE = NS * RPS

_EPS_BN = 1e-5
_EPS_NORM = 1e-12

_mesh = plsc.VectorSubcoreMesh(core_axis_name="c", subcore_axis_name="s")


def _zero_stripe(zfeat, accum, s):
    base0 = s * RPS
    pltpu.sync_copy(zfeat.at[pl.ds(base0, RPS)], accum.at[pl.ds(base0, RPS)])

    @pl.when(s == NS - 1)
    def _tail():
        pltpu.sync_copy(zfeat.at[pl.ds(TAIL_BASE, TAIL)],
                        accum.at[pl.ds(TAIL_BASE, TAIL)])


def _write_stripe(accum, out, c, s):
    base0 = s * RPS
    pltpu.sync_copy(accum.at[pl.ds(base0, RPS)],
                    out.at[c, pl.ds(base0, RPS)])

    @pl.when(s == NS - 1)
    def _tail():
        pltpu.sync_copy(accum.at[pl.ds(TAIL_BASE, TAIL)],
                        out.at[c, pl.ds(TAIL_BASE, TAIL)])


def _edge_loop(table, src, dst, wbase, sidx, didx, rows, sems, accum):
    def load_idx(k, b):
        base = wbase + k * CHUNK
        pltpu.sync_copy(src.at[pl.ds(base, CHUNK)], sidx[b])
        pltpu.sync_copy(dst.at[pl.ds(base, CHUNK)], didx[b])

    def gather(b):
        pltpu.async_copy(table.at[sidx[b]], rows[b], sems[b])

    def finish(b):
        pltpu.make_async_copy(table.at[sidx[b]], rows[b], sems[b]).wait()
        pltpu.sync_copy(rows[b], accum.at[didx[b]], add=True)

    def triad(j0, carry):
        k0 = 3 * j0
        for b in range(3):
            bp = (b + 2) % 3
            load_idx(k0 + b + 2, bp)
            gather(bp)
            finish(b)
        return carry

    load_idx(0, 0)
    gather(0)
    load_idx(1, 1)
    gather(1)
    lax.fori_loop(0, (ITERS - 2) // 3, triad, 0)
    finish((ITERS - 2) % 3)
    finish((ITERS - 1) % 3)


@functools.partial(
    pl.kernel,
    out_type=[
        jax.ShapeDtypeStruct((NC, N, D), jnp.float32),
        jax.ShapeDtypeStruct((NC, N, D), jnp.float32),
    ],
    mesh=_mesh,
    scratch_types=[
        pltpu.VMEM((CHUNK,), jnp.int32),
        pltpu.VMEM((CHUNK,), jnp.int32),
        pltpu.VMEM((CHUNK,), jnp.int32),
        pltpu.VMEM((CHUNK,), jnp.int32),
        pltpu.VMEM((CHUNK,), jnp.int32),
        pltpu.VMEM((CHUNK,), jnp.int32),
        pltpu.VMEM((CHUNK, D), jnp.float32),
        pltpu.VMEM((CHUNK, D), jnp.float32),
        pltpu.VMEM((CHUNK, D), jnp.float32),
        pltpu.VMEM_SHARED((N, D), jnp.float32),
        pltpu.SemaphoreType.DMA,
        pltpu.SemaphoreType.DMA,
        pltpu.SemaphoreType.DMA,
    ],
)
def _agg_counts(table, src, dst, zfeat, ones, sums_out, cnt_out,
                sidx0, sidx1, sidx2, didx0, didx1, didx2,
                rows0, rows1, rows2, accum, sem0, sem1, sem2):
    c = lax.axis_index("c")
    s = lax.axis_index("s")
    w = s * NC + c
    wbase = w * EPW

    _zero_stripe(zfeat, accum, s)
    pltpu.sync_copy(ones, rows0)
    plsc.subcore_barrier()

    def cscat(didx, sem):
        pltpu.async_copy(rows0, accum.at[didx], sem, add=True)

    def cwait(didx, sem):
        pltpu.make_async_copy(rows0, accum.at[didx], sem).wait()

    def cpair(j0, carry):
        k0 = 2 * j0
        pltpu.sync_copy(dst.at[pl.ds(wbase + (k0 + 1) * CHUNK, CHUNK)],
                        didx1)
        cscat(didx1, sem1)
        cwait(didx0, sem0)
        pltpu.sync_copy(dst.at[pl.ds(wbase + (k0 + 2) * CHUNK, CHUNK)],
                        didx0)
        cscat(didx0, sem0)
        cwait(didx1, sem1)
        return carry

    pltpu.sync_copy(dst.at[pl.ds(wbase, CHUNK)], didx0)
    cscat(didx0, sem0)
    lax.fori_loop(0, (ITERS - 1) // 2, cpair, 0)
    cwait(didx0, sem0)
    plsc.subcore_barrier()
    _write_stripe(accum, cnt_out, c, s)

    _zero_stripe(zfeat, accum, s)
    plsc.subcore_barrier()
    _edge_loop(table, src, dst, wbase, [sidx0, sidx1, sidx2],
               [didx0, didx1, didx2], [rows0, rows1, rows2],
               [sem0, sem1, sem2], accum)
    plsc.subcore_barrier()
    _write_stripe(accum, sums_out, c, s)


@functools.partial(
    pl.kernel,
    out_type=jax.ShapeDtypeStruct((NC, N, D), jnp.float32),
    mesh=_mesh,
    scratch_types=[
        pltpu.VMEM((CHUNK,), jnp.int32),
        pltpu.VMEM((CHUNK,), jnp.int32),
        pltpu.VMEM((CHUNK,), jnp.int32),
        pltpu.VMEM((CHUNK,), jnp.int32),
        pltpu.VMEM((CHUNK,), jnp.int32),
        pltpu.VMEM((CHUNK,), jnp.int32),
        pltpu.VMEM((CHUNK, D), jnp.float32),
        pltpu.VMEM((CHUNK, D), jnp.float32),
        pltpu.VMEM((CHUNK, D), jnp.float32),
        pltpu.VMEM_SHARED((N, D), jnp.float32),
        pltpu.SemaphoreType.DMA,
        pltpu.SemaphoreType.DMA,
        pltpu.SemaphoreType.DMA,
    ],
)
def _agg(table, src, dst, zfeat, sums_out,
         sidx0, sidx1, sidx2, didx0, didx1, didx2,
         rows0, rows1, rows2, accum, sem0, sem1, sem2):
    c = lax.axis_index("c")
    s = lax.axis_index("s")
    w = s * NC + c
    _zero_stripe(zfeat, accum, s)
    plsc.subcore_barrier()
    _edge_loop(table, src, dst, w * EPW, [sidx0, sidx1, sidx2],
               [didx0, didx1, didx2], [rows0, rows1, rows2],
               [sem0, sem1, sem2], accum)
    plsc.subcore_barrier()
    _write_stripe(accum, sums_out, c, s)


_R = 1000


def _dense1_body(sp_ref, cp_ref, x_ref, wl_ref, bl_ref, wr_ref, g_ref, b_ref,
                 o_ref):
    ssum = sp_ref[0] + sp_ref[1]
    cnt = cp_ref[0][:, 0:1] + cp_ref[1][:, 0:1]
    mean = ssum / jnp.maximum(cnt, 1.0)
    out = (jnp.dot(mean, wl_ref[...], preferred_element_type=jnp.float32)
           + jnp.dot(x_ref[...], wr_ref[...], preferred_element_type=jnp.float32)
           + bl_ref[...])
    nrm = jnp.sqrt(jnp.sum(out * out, axis=1, keepdims=True))
    out = out / jnp.maximum(nrm, _EPS_NORM)
    out = jnp.maximum(out, 0.0)
    o_ref[...] = g_ref[...] * out * (1.0 / jnp.sqrt(1.0 + _EPS_BN)) + b_ref[...]


def _dense2_body(sp_ref, cp_ref, h_ref, wl_ref, bl_ref, wr_ref, wfc_ref,
                 bfc_ref, o_ref):
    ssum = sp_ref[0] + sp_ref[1]
    cnt = cp_ref[0][:, 0:1] + cp_ref[1][:, 0:1]
    mean = ssum / jnp.maximum(cnt, 1.0)
    out = (jnp.dot(mean, wl_ref[...], preferred_element_type=jnp.float32)
           + jnp.dot(h_ref[...], wr_ref[...], preferred_element_type=jnp.float32)
           + bl_ref[...])
    nrm = jnp.sqrt(jnp.sum(out * out, axis=1, keepdims=True))
    out = out / jnp.maximum(nrm, _EPS_NORM)
    o_ref[...] = (jnp.sum(out * wfc_ref[...], axis=1, keepdims=True)
                  + bfc_ref[...])


def _row_specs():
    return [
        pl.BlockSpec((NC, _R, D), lambda i: (0, i, 0)),
        pl.BlockSpec((NC, _R, D), lambda i: (0, i, 0)),
        pl.BlockSpec((_R, D), lambda i: (i, 0)),
    ]


def _full2d(shape):
    return pl.BlockSpec(shape, lambda i: (0, 0))


def _dense1(sp, cp, x, wl, bl, wr, g, b):
    return pl.pallas_call(
        _dense1_body,
        grid=(N // _R,),
        in_specs=_row_specs() + [
            _full2d((D, D)), _full2d((1, D)), _full2d((D, D)),
            _full2d((1, D)), _full2d((1, D)),
        ],
        out_specs=pl.BlockSpec((_R, D), lambda i: (i, 0)),
        out_shape=jax.ShapeDtypeStruct((N, D), jnp.float32),
    )(sp, cp, x, wl, bl, wr, g, b)


def _dense2(sp, cp, h, wl, bl, wr, wfc, bfc):
    return pl.pallas_call(
        _dense2_body,
        grid=(N // _R,),
        in_specs=_row_specs() + [
            _full2d((D, D)), _full2d((1, D)), _full2d((D, D)),
            _full2d((1, D)), _full2d((1, 1)),
        ],
        out_specs=pl.BlockSpec((_R, 1), lambda i: (i, 0)),
        out_shape=jax.ShapeDtypeStruct((N, 1), jnp.float32),
    )(sp, cp, h, wl, bl, wr, wfc, bfc)


def kernel(x, edge_index, W1l, b1l, W1r, gamma, beta, W2l, b2l, W2r, Wfc, bfc):
    src = edge_index[0]
    dst = edge_index[1]
    zfeat = jnp.zeros((N, D), jnp.float32)
    ones = jnp.ones((CHUNK, D), jnp.float32)

    sums1, cnts = _agg_counts(x, src, dst, zfeat, ones)
    h = _dense1(sums1, cnts, x, W1l, b1l.reshape(1, D), W1r,
                gamma.reshape(1, D), beta.reshape(1, D))
    sums2 = _agg(h, src, dst, zfeat)
    out = _dense2(sums2, cnts, h, W2l, b2l.reshape(1, D), W2r,
                  Wfc.reshape(1, D), bfc.reshape(1, 1))
    return out.reshape(N)


# speedup vs baseline: 2.9272x; 1.2361x over previous
import functools

import jax
import jax.numpy as jnp
from jax import lax
from jax.experimental import pallas as pl
from jax.experimental.pallas import tpu as pltpu
from jax.experimental.pallas import tpu_sc as plsc

N = 10000
E = 320000
D = 128
NC = 2
NS = 16
NW = NC * NS
EPW = E // NW
CHUNK = 80
ITERS = EPW // CHUNK
RPS = 624
TAIL = N - NS * RPS
TAIL_BASE = NS * RPS

_EPS_BN = 1e-5
_EPS_NORM = 1e-12

_mesh = plsc.VectorSubcoreMesh(core_axis_name="c", subcore_axis_name="s")


def _zero_stripe(zfeat, accum, s):
    base0 = s * RPS
    pltpu.sync_copy(zfeat.at[pl.ds(base0, RPS)], accum.at[pl.ds(base0, RPS)])

    @pl.when(s == NS - 1)
    def _tail():
        pltpu.sync_copy(zfeat.at[pl.ds(TAIL_BASE, TAIL)],
                        accum.at[pl.ds(TAIL_BASE, TAIL)])


def _write_stripe(accum, out, c, s):
    base0 = s * RPS
    pltpu.sync_copy(accum.at[pl.ds(base0, RPS)],
                    out.at[c, pl.ds(base0, RPS)])

    @pl.when(s == NS - 1)
    def _tail():
        pltpu.sync_copy(accum.at[pl.ds(TAIL_BASE, TAIL)],
                        out.at[c, pl.ds(TAIL_BASE, TAIL)])


def _edge_loop(table, src, dst, wbase, sidx, didx, rows, gsems, ssems, accum):
    def load_idx(k, b):
        base = wbase + k * CHUNK
        pltpu.sync_copy(src.at[pl.ds(base, CHUNK)], sidx[b])
        pltpu.sync_copy(dst.at[pl.ds(base, CHUNK)], didx[b])

    def gather(b):
        pltpu.async_copy(table.at[sidx[b]], rows[b], gsems[b])

    def wait_g(b):
        pltpu.make_async_copy(table.at[sidx[b]], rows[b], gsems[b]).wait()

    def sc_issue(b):
        pltpu.async_copy(rows[b], accum.at[didx[b]], ssems[b], add=True)

    def sc_wait(b):
        pltpu.make_async_copy(rows[b], accum.at[didx[b]], ssems[b]).wait()

    load_idx(0, 0)
    gather(0)
    load_idx(1, 1)
    gather(1)
    for k in range(4):
        bp = (k + 2) % 4
        if k >= 2:
            sc_wait(bp)
        load_idx(k + 2, bp)
        gather(bp)
        wait_g(k)
        sc_issue(k)

    def quad(j0, carry):
        q = 4 + 4 * j0
        for b in range(4):
            k = q + b
            bp = (b + 2) % 4
            sc_wait(bp)
            load_idx(k + 2, bp)
            gather(bp)
            wait_g(b)
            sc_issue(b)
        return carry

    lax.fori_loop(0, (ITERS - 9) // 4, quad, 0)
    for k in range(ITERS - 5, ITERS - 2):
        bp = (k + 2) % 4
        sc_wait(bp)
        load_idx(k + 2, bp)
        gather(bp)
        wait_g(k % 4)
        sc_issue(k % 4)
    for k in range(ITERS - 2, ITERS):
        wait_g(k % 4)
        sc_issue(k % 4)
    for k in range(ITERS - 4, ITERS):
        sc_wait(k % 4)


@functools.partial(
    pl.kernel,
    out_type=[
        jax.ShapeDtypeStruct((NC, N, D), jnp.float32),
        jax.ShapeDtypeStruct((NC, N, D), jnp.float32),
    ],
    mesh=_mesh,
    scratch_types=[
        [pltpu.VMEM((CHUNK,), jnp.int32)] * 4,
        [pltpu.VMEM((CHUNK,), jnp.int32)] * 4,
        [pltpu.VMEM((CHUNK, D), jnp.float32)] * 4,
        pltpu.VMEM_SHARED((N, D), jnp.float32),
        [pltpu.SemaphoreType.DMA] * 4,
        [pltpu.SemaphoreType.DMA] * 4,
    ],
)
def _agg_counts(table, src, dst, zfeat, ones, sums_out, cnt_out,
                sidx, didx, rows, accum, gsems, ssems):
    c = lax.axis_index("c")
    s = lax.axis_index("s")
    w = s * NC + c
    wbase = w * EPW

    _zero_stripe(zfeat, accum, s)
    pltpu.sync_copy(ones, rows[0])
    plsc.subcore_barrier()

    def cscat(b):
        pltpu.async_copy(rows[0], accum.at[didx[b]], ssems[b], add=True)

    def cwait(b):
        pltpu.make_async_copy(rows[0], accum.at[didx[b]], ssems[b]).wait()

    def cload(k, b):
        pltpu.sync_copy(dst.at[pl.ds(wbase + k * CHUNK, CHUNK)], didx[b])

    def cpair(j0, carry):
        k0 = 2 * j0
        cload(k0 + 1, 1)
        cscat(1)
        cwait(0)
        cload(k0 + 2, 0)
        cscat(0)
        cwait(1)
        return carry

    cload(0, 0)
    cscat(0)
    lax.fori_loop(0, (ITERS - 1) // 2, cpair, 0)
    cwait(0)
    plsc.subcore_barrier()
    _write_stripe(accum, cnt_out, c, s)

    _zero_stripe(zfeat, accum, s)
    plsc.subcore_barrier()
    _edge_loop(table, src, dst, wbase, sidx, didx, rows, gsems, ssems, accum)
    plsc.subcore_barrier()
    _write_stripe(accum, sums_out, c, s)


@functools.partial(
    pl.kernel,
    out_type=jax.ShapeDtypeStruct((NC, N, D), jnp.float32),
    mesh=_mesh,
    scratch_types=[
        [pltpu.VMEM((CHUNK,), jnp.int32)] * 4,
        [pltpu.VMEM((CHUNK,), jnp.int32)] * 4,
        [pltpu.VMEM((CHUNK, D), jnp.float32)] * 4,
        pltpu.VMEM_SHARED((N, D), jnp.float32),
        [pltpu.SemaphoreType.DMA] * 4,
        [pltpu.SemaphoreType.DMA] * 4,
    ],
)
def _agg(table, src, dst, zfeat, sums_out,
         sidx, didx, rows, accum, gsems, ssems):
    c = lax.axis_index("c")
    s = lax.axis_index("s")
    w = s * NC + c
    _zero_stripe(zfeat, accum, s)
    plsc.subcore_barrier()
    _edge_loop(table, src, dst, w * EPW, sidx, didx, rows, gsems, ssems,
               accum)
    plsc.subcore_barrier()
    _write_stripe(accum, sums_out, c, s)


_R = 1000


def _dense1_body(sp_ref, cp_ref, x_ref, wl_ref, bl_ref, wr_ref, g_ref, b_ref,
                 o_ref):
    ssum = sp_ref[0] + sp_ref[1]
    cnt = cp_ref[0][:, 0:1] + cp_ref[1][:, 0:1]
    mean = ssum / jnp.maximum(cnt, 1.0)
    out = (jnp.dot(mean, wl_ref[...], preferred_element_type=jnp.float32)
           + jnp.dot(x_ref[...], wr_ref[...], preferred_element_type=jnp.float32)
           + bl_ref[...])
    nrm = jnp.sqrt(jnp.sum(out * out, axis=1, keepdims=True))
    out = out / jnp.maximum(nrm, _EPS_NORM)
    out = jnp.maximum(out, 0.0)
    o_ref[...] = g_ref[...] * out * (1.0 / jnp.sqrt(1.0 + _EPS_BN)) + b_ref[...]


def _dense2_body(sp_ref, cp_ref, h_ref, wl_ref, bl_ref, wr_ref, wfc_ref,
                 bfc_ref, o_ref):
    ssum = sp_ref[0] + sp_ref[1]
    cnt = cp_ref[0][:, 0:1] + cp_ref[1][:, 0:1]
    mean = ssum / jnp.maximum(cnt, 1.0)
    out = (jnp.dot(mean, wl_ref[...], preferred_element_type=jnp.float32)
           + jnp.dot(h_ref[...], wr_ref[...], preferred_element_type=jnp.float32)
           + bl_ref[...])
    nrm = jnp.sqrt(jnp.sum(out * out, axis=1, keepdims=True))
    out = out / jnp.maximum(nrm, _EPS_NORM)
    o_ref[...] = (jnp.sum(out * wfc_ref[...], axis=1, keepdims=True)
                  + bfc_ref[...])


def _row_specs():
    return [
        pl.BlockSpec((NC, _R, D), lambda i: (0, i, 0)),
        pl.BlockSpec((NC, _R, D), lambda i: (0, i, 0)),
        pl.BlockSpec((_R, D), lambda i: (i, 0)),
    ]


def _full2d(shape):
    return pl.BlockSpec(shape, lambda i: (0, 0))


def _dense1(sp, cp, x, wl, bl, wr, g, b):
    return pl.pallas_call(
        _dense1_body,
        grid=(N // _R,),
        in_specs=_row_specs() + [
            _full2d((D, D)), _full2d((1, D)), _full2d((D, D)),
            _full2d((1, D)), _full2d((1, D)),
        ],
        out_specs=pl.BlockSpec((_R, D), lambda i: (i, 0)),
        out_shape=jax.ShapeDtypeStruct((N, D), jnp.float32),
    )(sp, cp, x, wl, bl, wr, g, b)


def _dense2(sp, cp, h, wl, bl, wr, wfc, bfc):
    return pl.pallas_call(
        _dense2_body,
        grid=(N // _R,),
        in_specs=_row_specs() + [
            _full2d((D, D)), _full2d((1, D)), _full2d((D, D)),
            _full2d((1, D)), _full2d((1, 1)),
        ],
        out_specs=pl.BlockSpec((_R, 1), lambda i: (i, 0)),
        out_shape=jax.ShapeDtypeStruct((N, 1), jnp.float32),
    )(sp, cp, h, wl, bl, wr, wfc, bfc)


def kernel(x, edge_index, W1l, b1l, W1r, gamma, beta, W2l, b2l, W2r, Wfc, bfc):
    src = edge_index[0]
    dst = edge_index[1]
    zfeat = jnp.zeros((N, D), jnp.float32)
    ones = jnp.ones((CHUNK, D), jnp.float32)

    sums1, cnts = _agg_counts(x, src, dst, zfeat, ones)
    h = _dense1(sums1, cnts, x, W1l, b1l.reshape(1, D), W1r,
                gamma.reshape(1, D), beta.reshape(1, D))
    sums2 = _agg(h, src, dst, zfeat)
    out = _dense2(sums2, cnts, h, W2l, b2l.reshape(1, D), W2r,
                  Wfc.reshape(1, D), bfc.reshape(1, 1))
    return out.reshape(N)
